# baseline (speedup 1.0000x reference)
import functools

import jax
import jax.numpy as jnp
import numpy as np
from jax.experimental import pallas as pl
from jax.experimental.pallas import tpu as pltpu

MASK_RATIO = 0.3
FILL = 0.0


def _mask_kernel(smap_ref, mask_ref, *, k):
    gb, h, w = smap_ref.shape
    s = smap_ref[...]
    n = jnp.full((gb, 1, 1), h * w, jnp.int32)

    def count_ge_of(tf):
        return jnp.sum((s >= tf).astype(jnp.int32), axis=(1, 2),
                       keepdims=True)

    def write_mask(t, count_ge):
        tf = jax.lax.bitcast_convert_type(t, jnp.float32)
        mask_ref[...] = s < tf

        any_tie_split = jnp.sum((count_ge != k).astype(jnp.int32)) > 0

        @pl.when(any_tie_split)
        def _ties():
            eq = s == tf
            eqf = eq.astype(jnp.float32)
            a = jax.lax.broadcasted_iota(jnp.int32, (w, w), 0)
            b = jax.lax.broadcasted_iota(jnp.int32, (w, w), 1)
            su = (a < b).astype(jnp.float32)
            inrow = jax.lax.dot(eqf.reshape(gb * h, w),
                                su).reshape(gb, h, w)
            ah = jax.lax.broadcasted_iota(jnp.int32, (h, h), 0)
            bh = jax.lax.broadcasted_iota(jnp.int32, (h, h), 1)
            suh = (ah < bh).astype(jnp.float32)
            row_sums = jnp.sum(eqf, axis=2)
            row_pre = jax.lax.dot(row_sums, suh)
            rank = row_pre[:, :, None] + inrow
            count_gt = jnp.sum((s > tf).astype(jnp.float32),
                               axis=(1, 2), keepdims=True)
            rem = jnp.float32(k) - count_gt
            tie_rm = jnp.logical_and(eq, rank < rem)
            mask_ref[...] = jnp.logical_not(
                jnp.logical_or(s > tf, tie_rm))

    def bis_body(j, carry):
        lo, hi, cnt = carry
        mid = lo + ((hi - lo) >> 1)
        mf = jax.lax.bitcast_convert_type(mid, jnp.float32)
        c = count_ge_of(mf)
        ge = c >= k
        return (jnp.where(ge, mid, lo), jnp.where(ge, hi, mid),
                jnp.where(ge, c, cnt))

    lo0 = jnp.int32(0x3F30A3D7)
    hi0 = jnp.int32(0x3F35C28F)
    cnt_lo = count_ge_of(jnp.float32(0.69))
    cnt_hi = count_ge_of(jnp.float32(0.71))
    ok = jnp.logical_and(cnt_lo >= k, cnt_hi < k)
    all_ok = jnp.sum(ok.astype(jnp.int32)) == gb

    @pl.when(all_ok)
    def _fast():
        lo = jnp.full((gb, 1, 1), lo0)
        hi = jnp.full((gb, 1, 1), hi0)
        t, _, count_ge = jax.lax.fori_loop(
            0, 19, bis_body, (lo, hi, cnt_lo))
        write_mask(t, count_ge)

    @pl.when(jnp.logical_not(all_ok))
    def _exact_full_range():
        lo = jnp.zeros((gb, 1, 1), jnp.int32)
        hi = jnp.full((gb, 1, 1), jnp.int32(1 << 30))
        t, _, count_ge = jax.lax.fori_loop(
            0, 30, bis_body, (lo, hi, n))
        write_mask(t, count_ge)


def _fill_kernel(mask_ref, img_ref, out_ref):
    out_ref[...] = jnp.where(mask_ref[...][:, None], img_ref[...],
                             jnp.float32(FILL))


def kernel(img, smap):
    B, C, H, W = img.shape
    k = int(round(MASK_RATIO * H * W))
    GB = 4 if B % 4 == 0 else 1
    mask = pl.pallas_call(
        functools.partial(_mask_kernel, k=k),
        grid=(B // GB,),
        in_specs=[pl.BlockSpec((GB, H, W), lambda b: (b, 0, 0))],
        out_specs=pl.BlockSpec((GB, H, W), lambda b: (b, 0, 0)),
        out_shape=jax.ShapeDtypeStruct((B, H, W), jnp.bool_),
    )(smap)
    FB = 2 if B % 2 == 0 else 1
    out = pl.pallas_call(
        _fill_kernel,
        grid=(B // FB,),
        in_specs=[
            pl.BlockSpec((FB, H, W), lambda b: (b, 0, 0)),
            pl.BlockSpec((FB, C, H, W), lambda b: (b, 0, 0, 0)),
        ],
        out_specs=pl.BlockSpec((FB, C, H, W), lambda b: (b, 0, 0, 0)),
        out_shape=jax.ShapeDtypeStruct((B, C, H, W), jnp.float32),
    )(mask, img)
    return out, mask


# speedup vs baseline: 1.1019x; 1.1019x over previous
import functools

import jax
import jax.numpy as jnp
import numpy as np
from jax.experimental import pallas as pl
from jax.experimental.pallas import tpu as pltpu

MASK_RATIO = 0.3
FILL = 0.0


def _mask_kernel(smap_ref, mask_ref, *, k):
    gb, h, w = smap_ref.shape
    s = smap_ref[...]
    n = jnp.full((gb, 1, 1), h * w, jnp.int32)

    def count_ge_of(tf):
        return jnp.sum((s >= tf).astype(jnp.int32), axis=(1, 2),
                       keepdims=True)

    def write_mask(t, count_ge):
        tf = jax.lax.bitcast_convert_type(t, jnp.float32)
        mask_ref[...] = s < tf

        any_tie_split = jnp.sum((count_ge != k).astype(jnp.int32)) > 0

        @pl.when(any_tie_split)
        def _ties():
            eq = s == tf
            eqf = eq.astype(jnp.float32)
            a = jax.lax.broadcasted_iota(jnp.int32, (w, w), 0)
            b = jax.lax.broadcasted_iota(jnp.int32, (w, w), 1)
            su = (a < b).astype(jnp.float32)
            inrow = jax.lax.dot(eqf.reshape(gb * h, w),
                                su).reshape(gb, h, w)
            ah = jax.lax.broadcasted_iota(jnp.int32, (h, h), 0)
            bh = jax.lax.broadcasted_iota(jnp.int32, (h, h), 1)
            suh = (ah < bh).astype(jnp.float32)
            row_sums = jnp.sum(eqf, axis=2)
            row_pre = jax.lax.dot(row_sums, suh)
            rank = row_pre[:, :, None] + inrow
            count_gt = jnp.sum((s > tf).astype(jnp.float32),
                               axis=(1, 2), keepdims=True)
            rem = jnp.float32(k) - count_gt
            tie_rm = jnp.logical_and(eq, rank < rem)
            mask_ref[...] = jnp.logical_not(
                jnp.logical_or(s > tf, tie_rm))

    def bis_body(j, carry):
        lo, hi, cnt = carry
        mid = lo + ((hi - lo) >> 1)
        mf = jax.lax.bitcast_convert_type(mid, jnp.float32)
        c = count_ge_of(mf)
        ge = c >= k
        return (jnp.where(ge, mid, lo), jnp.where(ge, hi, mid),
                jnp.where(ge, c, cnt))

    lo0 = jnp.int32(0x3F30A3D7)
    hi0 = jnp.int32(0x3F35C28F)
    cnt_lo = count_ge_of(jnp.float32(0.69))
    cnt_hi = count_ge_of(jnp.float32(0.71))
    ok = jnp.logical_and(cnt_lo >= k, cnt_hi < k)
    all_ok = jnp.sum(ok.astype(jnp.int32)) == gb

    @pl.when(all_ok)
    def _fast():
        lo = jnp.full((gb, 1, 1), lo0)
        hi = jnp.full((gb, 1, 1), hi0)
        t, _, count_ge = jax.lax.fori_loop(
            0, 19, bis_body, (lo, hi, cnt_lo))
        write_mask(t, count_ge)

    @pl.when(jnp.logical_not(all_ok))
    def _exact_full_range():
        lo = jnp.zeros((gb, 1, 1), jnp.int32)
        hi = jnp.full((gb, 1, 1), jnp.int32(1 << 30))
        t, _, count_ge = jax.lax.fori_loop(
            0, 30, bis_body, (lo, hi, n))
        write_mask(t, count_ge)


def _fill_kernel(mask_ref, img_ref, out_ref):
    out_ref[...] = jnp.where(mask_ref[...][:, None], img_ref[...],
                             jnp.float32(FILL))


def kernel(img, smap):
    B, C, H, W = img.shape
    k = int(round(MASK_RATIO * H * W))
    GB = 8 if B % 8 == 0 else 1
    mask = pl.pallas_call(
        functools.partial(_mask_kernel, k=k),
        grid=(B // GB,),
        in_specs=[pl.BlockSpec((GB, H, W), lambda b: (b, 0, 0))],
        out_specs=pl.BlockSpec((GB, H, W), lambda b: (b, 0, 0)),
        out_shape=jax.ShapeDtypeStruct((B, H, W), jnp.bool_),
    )(smap)
    FB = 2 if B % 2 == 0 else 1
    out = pl.pallas_call(
        _fill_kernel,
        grid=(B // FB,),
        in_specs=[
            pl.BlockSpec((FB, H, W), lambda b: (b, 0, 0)),
            pl.BlockSpec((FB, C, H, W), lambda b: (b, 0, 0, 0)),
        ],
        out_specs=pl.BlockSpec((FB, C, H, W), lambda b: (b, 0, 0, 0)),
        out_shape=jax.ShapeDtypeStruct((B, C, H, W), jnp.float32),
    )(mask, img)
    return out, mask
